# baseline (speedup 1.0000x reference)
import functools

import jax
import jax.numpy as jnp
from jax import lax
from jax.experimental import pallas as pl
from jax.experimental.pallas import tpu as pltpu
from jax.experimental.pallas import tpu_sc as plsc

N_NODES = 10000
D = 128
N_EDGES = 320000
NC, NS = 2, 16
NW = NC * NS
GROUP = 128
G0 = 104
G1 = 56
TOTAL_GROUPS = NS * (G0 + G1)
E_PAD = TOTAL_GROUPS * GROUP
ROWS_PER_TILE = 632
N_PAD = NS * ROWS_PER_TILE
PAD_SRC = N_NODES

_mesh = plsc.VectorSubcoreMesh(
    core_axis_name="c", subcore_axis_name="s", num_cores=NC, num_subcores=NS
)


@functools.partial(
    pl.kernel,
    out_type=jax.ShapeDtypeStruct((NC, N_PAD, D), jnp.float32),
    mesh=_mesh,
    scratch_types=[
        pltpu.VMEM((G0, GROUP), jnp.int32),
        pltpu.VMEM((G0, GROUP), jnp.int32),
        pltpu.VMEM((GROUP, D), jnp.float32),
        pltpu.VMEM_SHARED((N_PAD, D), jnp.float32),
        pltpu.SemaphoreType.DMA,
    ],
)
def _gcn_aggregate(feat_hbm, feat2_hbm, src_hbm, dst_hbm, zeros_hbm, out_hbm,
                   idx_s, idx_d, rows, hagg, sem):
    cid = lax.axis_index("c")
    sid = lax.axis_index("s")
    wid = sid * NC + cid

    with jax.named_scope("agg_init"):
        pltpu.sync_copy(
            zeros_hbm.at[pl.ds(sid * ROWS_PER_TILE, ROWS_PER_TILE)],
            hagg.at[pl.ds(sid * ROWS_PER_TILE, ROWS_PER_TILE)])
        pltpu.sync_copy(src_hbm.at[wid], idx_s)
        pltpu.sync_copy(dst_hbm.at[wid], idx_d)
        plsc.subcore_barrier()

    with jax.named_scope("agg_edges"):
        def run(feat_ref, n_groups):
            def body(j, carry):
                pltpu.async_copy(feat_ref.at[idx_s.at[j]], rows, sem).wait()
                pltpu.sync_copy(rows, hagg.at[idx_d.at[j]], add=True)
                return carry

            lax.fori_loop(0, n_groups, body, 0)

        @pl.when(cid == 0)
        def _():
            run(feat_hbm, G0)

        @pl.when(cid == 1)
        def _():
            run(feat2_hbm, G1)

        plsc.subcore_barrier()

    with jax.named_scope("agg_writeout"):
        pltpu.sync_copy(
            hagg.at[pl.ds(sid * ROWS_PER_TILE, ROWS_PER_TILE)],
            out_hbm.at[cid, pl.ds(sid * ROWS_PER_TILE, ROWS_PER_TILE)])


def _linear_relu_body(parts_ref, wt_ref, b_ref, o_ref):
    x = parts_ref[0] + parts_ref[1]
    y = jnp.dot(x, wt_ref[...], preferred_element_type=jnp.float32)
    o_ref[...] = jnp.maximum(y + b_ref[...], 0.0)


_BLK = N_PAD // 8


def _apply_linear_relu(parts, wt, b2):
    return pl.pallas_call(
        _linear_relu_body,
        grid=(N_PAD // _BLK,),
        in_specs=[
            pl.BlockSpec((NC, _BLK, D), lambda i: (0, i, 0)),
            pl.BlockSpec((D, D), lambda i: (0, 0)),
            pl.BlockSpec((1, D), lambda i: (0, 0)),
        ],
        out_specs=pl.BlockSpec((_BLK, D), lambda i: (i, 0)),
        out_shape=jax.ShapeDtypeStruct((N_PAD, D), jnp.float32),
    )(parts, wt, b2)


@jax.jit
def kernel(feature, edge_index, W, b):
    src = edge_index[0].astype(jnp.int32)
    dst = edge_index[1].astype(jnp.int32)
    pad = E_PAD - N_EDGES

    def layout(flat):
        g = flat.reshape(TOTAL_GROUPS, GROUP)
        ev = g[: NS * G0].reshape(NS, G0, GROUP)
        od = g[NS * G0:].reshape(NS, G1, GROUP)
        od = jnp.concatenate(
            [od, jnp.zeros((NS, G0 - G1, GROUP), jnp.int32)], axis=1)
        return jnp.stack([ev, od], axis=1).reshape(NW, G0, GROUP)

    src_p = layout(jnp.concatenate(
        [src, jnp.full((pad,), PAD_SRC, jnp.int32)]))
    dst_p = layout(jnp.concatenate([dst, jnp.zeros((pad,), jnp.int32)]))
    feat_ext = jnp.concatenate(
        [feature, jnp.zeros((16, D), feature.dtype)], axis=0)
    feat_ext2 = jnp.concatenate(
        [lax.optimization_barrier(feature), jnp.zeros((16, D), feature.dtype)],
        axis=0)
    zeros = jnp.zeros((N_PAD, D), jnp.float32)
    parts = _gcn_aggregate(feat_ext, feat_ext2, src_p, dst_p, zeros)
    return _apply_linear_relu(parts, W.T, b.reshape(1, D))[:N_NODES]


# speedup vs baseline: 2.3985x; 2.3985x over previous
import functools

import jax
import jax.numpy as jnp
from jax import lax
from jax.experimental import pallas as pl
from jax.experimental.pallas import tpu as pltpu
from jax.experimental.pallas import tpu_sc as plsc

N_NODES = 10000
D = 128
N_EDGES = 320000
NC, NS = 2, 16
NW = NC * NS
GROUP = 128
G0 = 80
G1 = 80
TOTAL_GROUPS = NS * (G0 + G1)
E_PAD = TOTAL_GROUPS * GROUP
ROWS_PER_TILE = 632
N_PAD = NS * ROWS_PER_TILE
N_ZPAD = 128

_mesh = plsc.VectorSubcoreMesh(
    core_axis_name="c", subcore_axis_name="s", num_cores=NC, num_subcores=NS
)


@functools.partial(
    pl.kernel,
    out_type=jax.ShapeDtypeStruct((NC, N_PAD, D), jnp.float32),
    mesh=_mesh,
    scratch_types=[
        pltpu.VMEM((G0, GROUP), jnp.int32),
        pltpu.VMEM((G0, GROUP), jnp.int32),
        pltpu.VMEM((GROUP, D), jnp.float32),
        pltpu.VMEM_SHARED((N_PAD, D), jnp.float32),
        pltpu.SemaphoreType.DMA,
    ],
)
def _gcn_aggregate(feat_hbm, feat2_hbm, src_hbm, dst_hbm, zeros_hbm, out_hbm,
                   idx_s, idx_d, rows, hagg, sem):
    cid = lax.axis_index("c")
    sid = lax.axis_index("s")
    wid = sid * NC + cid

    with jax.named_scope("agg_init"):
        pltpu.sync_copy(
            zeros_hbm.at[pl.ds(sid * ROWS_PER_TILE, ROWS_PER_TILE)],
            hagg.at[pl.ds(sid * ROWS_PER_TILE, ROWS_PER_TILE)])
        pltpu.sync_copy(src_hbm.at[wid], idx_s)
        pltpu.sync_copy(dst_hbm.at[wid], idx_d)
        plsc.subcore_barrier()

    with jax.named_scope("agg_edges"):
        def run(feat_ref, n_groups):
            def body(j, carry):
                pltpu.async_copy(feat_ref.at[idx_s.at[j]], rows, sem).wait()
                pltpu.sync_copy(rows, hagg.at[idx_d.at[j]], add=True)
                return carry

            lax.fori_loop(0, n_groups, body, 0)

        @pl.when(cid == 0)
        def _():
            run(feat_hbm, G0)

        @pl.when(cid == 1)
        def _():
            run(feat2_hbm, G1)

        plsc.subcore_barrier()

    with jax.named_scope("agg_writeout"):
        pltpu.sync_copy(
            hagg.at[pl.ds(sid * ROWS_PER_TILE, ROWS_PER_TILE)],
            out_hbm.at[cid, pl.ds(sid * ROWS_PER_TILE, ROWS_PER_TILE)])


def _linear_relu_body(parts_ref, wt_ref, b_ref, o_ref):
    x = parts_ref[0] + parts_ref[1]
    y = jnp.dot(x, wt_ref[...], preferred_element_type=jnp.float32)
    o_ref[...] = jnp.maximum(y + b_ref[...], 0.0)


_BLK = N_PAD // 8


def _apply_linear_relu(parts, wt, b2):
    return pl.pallas_call(
        _linear_relu_body,
        grid=(N_PAD // _BLK,),
        in_specs=[
            pl.BlockSpec((NC, _BLK, D), lambda i: (0, i, 0)),
            pl.BlockSpec((D, D), lambda i: (0, 0)),
            pl.BlockSpec((1, D), lambda i: (0, 0)),
        ],
        out_specs=pl.BlockSpec((_BLK, D), lambda i: (i, 0)),
        out_shape=jax.ShapeDtypeStruct((N_PAD, D), jnp.float32),
    )(parts, wt, b2)


@jax.jit
def kernel(feature, edge_index, W, b):
    src = edge_index[0].astype(jnp.int32)
    dst = edge_index[1].astype(jnp.int32)
    pad = E_PAD - N_EDGES

    def layout(flat):
        g = flat.reshape(TOTAL_GROUPS, GROUP)
        ev = g[: NS * G0].reshape(NS, G0, GROUP)
        od = g[NS * G0:].reshape(NS, G1, GROUP)
        od = jnp.concatenate(
            [od, jnp.zeros((NS, G0 - G1, GROUP), jnp.int32)], axis=1)
        return jnp.stack([ev, od], axis=1).reshape(NW, G0, GROUP)

    spread = jnp.arange(pad, dtype=jnp.int32) % N_ZPAD
    src_p = layout(jnp.concatenate([src, N_NODES + spread]))
    dst_p = layout(jnp.concatenate([dst, spread]))
    zpad = jnp.zeros((N_ZPAD, D), feature.dtype)
    feat_ext = jnp.concatenate([feature, zpad], axis=0)
    feat_ext2 = jnp.concatenate(
        [lax.optimization_barrier(feature), zpad], axis=0)
    zeros = jnp.zeros((N_PAD, D), jnp.float32)
    parts = _gcn_aggregate(feat_ext, feat_ext2, src_p, dst_p, zeros)
    return _apply_linear_relu(parts, W.T, b.reshape(1, D))[:N_NODES]


# speedup vs baseline: 2.4574x; 1.0246x over previous
import functools

import jax
import jax.numpy as jnp
from jax import lax
from jax.experimental import pallas as pl
from jax.experimental.pallas import tpu as pltpu
from jax.experimental.pallas import tpu_sc as plsc

N_NODES = 10000
D = 128
N_EDGES = 320000
NC, NS = 2, 16
NW = NC * NS
GROUP = 128
G0 = 80
G1 = 80
TOTAL_GROUPS = NS * (G0 + G1)
E_PAD = TOTAL_GROUPS * GROUP
ROWS_PER_TILE = 632
N_PAD = NS * ROWS_PER_TILE
N_ZPAD = 128

_mesh = plsc.VectorSubcoreMesh(
    core_axis_name="c", subcore_axis_name="s", num_cores=NC, num_subcores=NS
)


@functools.partial(
    pl.kernel,
    out_type=jax.ShapeDtypeStruct((NC, N_PAD, D), jnp.float32),
    mesh=_mesh,
    scratch_types=[
        pltpu.VMEM((G0, GROUP), jnp.int32),
        pltpu.VMEM((G0, GROUP), jnp.int32),
        pltpu.VMEM((GROUP, D), jnp.float32),
        pltpu.VMEM_SHARED((N_PAD, D), jnp.float32),
        pltpu.SemaphoreType.DMA,
    ],
)
def _gcn_aggregate(feat_hbm, feat2_hbm, src_hbm, dst_hbm, zeros_hbm, out_hbm,
                   idx_s, idx_d, rows, hagg, sem):
    cid = lax.axis_index("c")
    sid = lax.axis_index("s")
    wid = sid * NC + cid

    with jax.named_scope("agg_init"):
        pltpu.sync_copy(
            zeros_hbm.at[pl.ds(sid * ROWS_PER_TILE, ROWS_PER_TILE)],
            hagg.at[pl.ds(sid * ROWS_PER_TILE, ROWS_PER_TILE)])
        pltpu.sync_copy(src_hbm.at[wid], idx_s)
        pltpu.sync_copy(dst_hbm.at[wid], idx_d)
        plsc.subcore_barrier()

    with jax.named_scope("agg_edges"):
        def run(feat_ref, n_groups):
            def body(j, carry):
                pltpu.async_copy(feat_ref.at[idx_s.at[j]], rows, sem).wait()
                pltpu.sync_copy(rows, hagg.at[idx_d.at[j]], add=True)
                return carry

            lax.fori_loop(0, n_groups, body, 0)

        @pl.when(cid == 0)
        def _():
            run(feat_hbm, G0)

        @pl.when(cid == 1)
        def _():
            run(feat2_hbm, G1)

        plsc.subcore_barrier()

    with jax.named_scope("agg_writeout"):
        pltpu.sync_copy(
            hagg.at[pl.ds(sid * ROWS_PER_TILE, ROWS_PER_TILE)],
            out_hbm.at[cid, pl.ds(sid * ROWS_PER_TILE, ROWS_PER_TILE)])


def _linear_relu_body(parts_ref, wt_ref, b_ref, o_ref):
    x = parts_ref[0] + parts_ref[1]
    y = jnp.dot(x, wt_ref[...], preferred_element_type=jnp.float32)
    o_ref[...] = jnp.maximum(y + b_ref[...], 0.0)


_BLK = 2000


def _apply_linear_relu(parts, wt, b2):
    return pl.pallas_call(
        _linear_relu_body,
        grid=(N_NODES // _BLK,),
        in_specs=[
            pl.BlockSpec((NC, _BLK, D), lambda i: (0, i, 0)),
            pl.BlockSpec((D, D), lambda i: (0, 0)),
            pl.BlockSpec((1, D), lambda i: (0, 0)),
        ],
        out_specs=pl.BlockSpec((_BLK, D), lambda i: (i, 0)),
        out_shape=jax.ShapeDtypeStruct((N_NODES, D), jnp.float32),
    )(parts, wt, b2)


@jax.jit
def kernel(feature, edge_index, W, b):
    src = edge_index[0].astype(jnp.int32)
    dst = edge_index[1].astype(jnp.int32)
    pad = E_PAD - N_EDGES

    def layout(flat):
        g = flat.reshape(TOTAL_GROUPS, GROUP)
        ev = g[: NS * G0].reshape(NS, G0, GROUP)
        od = g[NS * G0:].reshape(NS, G1, GROUP)
        od = jnp.concatenate(
            [od, jnp.zeros((NS, G0 - G1, GROUP), jnp.int32)], axis=1)
        return jnp.stack([ev, od], axis=1).reshape(NW, G0, GROUP)

    spread = jnp.arange(pad, dtype=jnp.int32) % N_ZPAD
    src_p = layout(jnp.concatenate([src, N_NODES + spread]))
    dst_p = layout(jnp.concatenate([dst, spread]))
    zpad = jnp.zeros((N_ZPAD, D), feature.dtype)
    feat_ext = jnp.concatenate([feature, zpad], axis=0)
    feat_ext2 = jnp.concatenate(
        [lax.optimization_barrier(feature), zpad], axis=0)
    zeros = jnp.zeros((N_PAD, D), jnp.float32)
    parts = _gcn_aggregate(feat_ext, feat_ext2, src_p, dst_p, zeros)
    return _apply_linear_relu(parts, W.T, b.reshape(1, D))


# speedup vs baseline: 3.1701x; 1.2900x over previous
import functools

import jax
import jax.numpy as jnp
from jax import lax
from jax.experimental import pallas as pl
from jax.experimental.pallas import tpu as pltpu
from jax.experimental.pallas import tpu_sc as plsc

N_NODES = 10000
D = 128
N_EDGES = 320000
NC, NS = 2, 16
NW = NC * NS
GROUP = 128
G = 80
E_PAD = NW * G * GROUP
ROWS_PER_TILE = 632
N_PAD = NS * ROWS_PER_TILE
N_ZPAD = 128

_mesh = plsc.VectorSubcoreMesh(
    core_axis_name="c", subcore_axis_name="s", num_cores=NC, num_subcores=NS
)


@functools.partial(
    pl.kernel,
    out_type=jax.ShapeDtypeStruct((NC, N_PAD, D), jnp.float32),
    mesh=_mesh,
    scratch_types=[
        pltpu.VMEM((G, GROUP), jnp.int32),
        pltpu.VMEM((GROUP,), jnp.int32),
        pltpu.VMEM((GROUP,), jnp.int32),
        pltpu.VMEM((GROUP,), jnp.int32),
        pltpu.VMEM((GROUP,), jnp.int32),
        pltpu.VMEM((GROUP, D), jnp.float32),
        pltpu.VMEM((GROUP, D), jnp.float32),
        pltpu.VMEM_SHARED((N_PAD, D), jnp.float32),
        pltpu.SemaphoreType.DMA,
        pltpu.SemaphoreType.DMA,
        pltpu.SemaphoreType.DMA,
        pltpu.SemaphoreType.DMA,
    ],
)
def _gcn_aggregate(feat_hbm, pk_hbm, zeros_hbm, out_hbm,
                   pk, sA, dA, sB, dB, rowsA, rowsB, hagg,
                   gsA, gsB, ssA, ssB):
    cid = lax.axis_index("c")
    sid = lax.axis_index("s")
    wid = sid * NC + cid

    with jax.named_scope("agg_init"):
        pltpu.sync_copy(
            zeros_hbm.at[pl.ds(sid * ROWS_PER_TILE, ROWS_PER_TILE)],
            hagg.at[pl.ds(sid * ROWS_PER_TILE, ROWS_PER_TILE)])
        pltpu.sync_copy(pk_hbm.at[wid], pk)
        plsc.subcore_barrier()

    def unpack(j, s_ref, d_ref):
        for t in range(GROUP // 16):
            w = pk[j, pl.ds(16 * t, 16)]
            s_ref[pl.ds(16 * t, 16)] = jnp.bitwise_and(w, 0xFFFF)
            d_ref[pl.ds(16 * t, 16)] = lax.shift_right_logical(w, 16)

    def g_start(s_ref, rows_ref, sem):
        pltpu.async_copy(feat_hbm.at[s_ref], rows_ref, sem)

    def g_wait(s_ref, rows_ref, sem):
        pltpu.make_async_copy(feat_hbm.at[s_ref], rows_ref, sem).wait()

    def s_start(d_ref, rows_ref, sem):
        pltpu.async_copy(rows_ref, hagg.at[d_ref], sem, add=True)

    def s_wait(d_ref, rows_ref, sem):
        pltpu.make_async_copy(rows_ref, hagg.at[d_ref], sem).wait()

    with jax.named_scope("agg_edges"):
        unpack(0, sA, dA)
        g_start(sA, rowsA, gsA)
        unpack(1, sB, dB)
        g_start(sB, rowsB, gsB)
        g_wait(sA, rowsA, gsA)
        s_start(dA, rowsA, ssA)

        def body(jj, carry):
            j = 2 * jj
            g_wait(sB, rowsB, gsB)
            s_wait(dA, rowsA, ssA)
            unpack(j, sA, dA)
            g_start(sA, rowsA, gsA)
            s_start(dB, rowsB, ssB)
            g_wait(sA, rowsA, gsA)
            s_wait(dB, rowsB, ssB)
            unpack(j + 1, sB, dB)
            g_start(sB, rowsB, gsB)
            s_start(dA, rowsA, ssA)
            return carry

        lax.fori_loop(1, G // 2, body, 0)

        g_wait(sB, rowsB, gsB)
        s_wait(dA, rowsA, ssA)
        s_start(dB, rowsB, ssB)
        s_wait(dB, rowsB, ssB)
        plsc.subcore_barrier()

    with jax.named_scope("agg_writeout"):
        pltpu.sync_copy(
            hagg.at[pl.ds(sid * ROWS_PER_TILE, ROWS_PER_TILE)],
            out_hbm.at[cid, pl.ds(sid * ROWS_PER_TILE, ROWS_PER_TILE)])


def _linear_relu_body(parts_ref, wt_ref, b_ref, o_ref):
    x = parts_ref[0] + parts_ref[1]
    y = jnp.dot(x, wt_ref[...], preferred_element_type=jnp.float32)
    o_ref[...] = jnp.maximum(y + b_ref[...], 0.0)


_BLK = 2000


def _apply_linear_relu(parts, wt, b2):
    return pl.pallas_call(
        _linear_relu_body,
        grid=(N_NODES // _BLK,),
        in_specs=[
            pl.BlockSpec((NC, _BLK, D), lambda i: (0, i, 0)),
            pl.BlockSpec((D, D), lambda i: (0, 0)),
            pl.BlockSpec((1, D), lambda i: (0, 0)),
        ],
        out_specs=pl.BlockSpec((_BLK, D), lambda i: (i, 0)),
        out_shape=jax.ShapeDtypeStruct((N_NODES, D), jnp.float32),
    )(parts, wt, b2)


@jax.jit
def kernel(feature, edge_index, W, b):
    src = edge_index[0].astype(jnp.int32)
    dst = edge_index[1].astype(jnp.int32)
    pad = E_PAD - N_EDGES
    spread = jnp.arange(pad, dtype=jnp.int32) % N_ZPAD
    src_p = jnp.concatenate([src, N_NODES + spread])
    dst_p = jnp.concatenate([dst, spread])
    packed = jnp.bitwise_or(
        src_p, lax.shift_left(dst_p, 16)).reshape(NW, G, GROUP)
    zpad = jnp.zeros((N_ZPAD, D), feature.dtype)
    feat_ext = jnp.concatenate([feature, zpad], axis=0)
    zeros = jnp.zeros((N_PAD, D), jnp.float32)
    parts = _gcn_aggregate(feat_ext, packed, zeros)
    return _apply_linear_relu(parts, W.T, b.reshape(1, D))


# speedup vs baseline: 3.2416x; 1.0225x over previous
import functools

import jax
import jax.numpy as jnp
from jax import lax
from jax.experimental import pallas as pl
from jax.experimental.pallas import tpu as pltpu
from jax.experimental.pallas import tpu_sc as plsc

N_NODES = 10000
D = 128
N_EDGES = 320000
NC, NS = 2, 16
NW = NC * NS
GROUP = 128
G = 80
E_PAD = NW * G * GROUP
ROWS_PER_TILE = 632
N_PAD = NS * ROWS_PER_TILE

_mesh = plsc.VectorSubcoreMesh(
    core_axis_name="c", subcore_axis_name="s", num_cores=NC, num_subcores=NS
)


@functools.partial(
    pl.kernel,
    out_type=jax.ShapeDtypeStruct((NC, N_PAD, D), jnp.float32),
    mesh=_mesh,
    scratch_types=[
        pltpu.VMEM((G, GROUP), jnp.int32),
        pltpu.VMEM((GROUP,), jnp.int32),
        pltpu.VMEM((GROUP,), jnp.int32),
        pltpu.VMEM((GROUP,), jnp.int32),
        pltpu.VMEM((GROUP,), jnp.int32),
        pltpu.VMEM((GROUP, D), jnp.float32),
        pltpu.VMEM((GROUP, D), jnp.float32),
        pltpu.VMEM_SHARED((N_PAD, D), jnp.float32),
        pltpu.SemaphoreType.DMA,
        pltpu.SemaphoreType.DMA,
        pltpu.SemaphoreType.DMA,
        pltpu.SemaphoreType.DMA,
    ],
)
def _gcn_aggregate(feat_hbm, pk_hbm, zeros_hbm, out_hbm,
                   pk, sA, dA, sB, dB, rowsA, rowsB, hagg,
                   gsA, gsB, ssA, ssB):
    cid = lax.axis_index("c")
    sid = lax.axis_index("s")
    wid = sid * NC + cid

    with jax.named_scope("agg_init"):
        pltpu.async_copy(pk_hbm.at[wid], pk, gsA)
        pltpu.sync_copy(
            zeros_hbm.at[pl.ds(sid * ROWS_PER_TILE, ROWS_PER_TILE)],
            hagg.at[pl.ds(sid * ROWS_PER_TILE, ROWS_PER_TILE)])
        pltpu.make_async_copy(pk_hbm.at[wid], pk, gsA).wait()
        plsc.subcore_barrier()

    def unpack(j, s_ref, d_ref):
        for t in range(GROUP // 16):
            w = pk[j, pl.ds(16 * t, 16)]
            s_ref[pl.ds(16 * t, 16)] = jnp.bitwise_and(w, 0xFFFF)
            d_ref[pl.ds(16 * t, 16)] = lax.shift_right_logical(w, 16)

    def g_start(s_ref, rows_ref, sem):
        pltpu.async_copy(feat_hbm.at[s_ref], rows_ref, sem)

    def g_wait(s_ref, rows_ref, sem):
        pltpu.make_async_copy(feat_hbm.at[s_ref], rows_ref, sem).wait()

    def s_start(d_ref, rows_ref, sem):
        pltpu.async_copy(rows_ref, hagg.at[d_ref], sem, add=True)

    def s_wait(d_ref, rows_ref, sem):
        pltpu.make_async_copy(rows_ref, hagg.at[d_ref], sem).wait()

    with jax.named_scope("agg_edges"):
        unpack(0, sA, dA)
        g_start(sA, rowsA, gsA)
        unpack(1, sB, dB)
        g_start(sB, rowsB, gsB)
        g_wait(sA, rowsA, gsA)
        s_start(dA, rowsA, ssA)

        def body(jj, carry):
            j = 2 * jj
            g_wait(sB, rowsB, gsB)
            s_wait(dA, rowsA, ssA)
            unpack(j, sA, dA)
            g_start(sA, rowsA, gsA)
            s_start(dB, rowsB, ssB)
            g_wait(sA, rowsA, gsA)
            s_wait(dB, rowsB, ssB)
            unpack(j + 1, sB, dB)
            g_start(sB, rowsB, gsB)
            s_start(dA, rowsA, ssA)
            return carry

        lax.fori_loop(1, G // 2, body, 0)

        g_wait(sB, rowsB, gsB)
        s_wait(dA, rowsA, ssA)
        s_start(dB, rowsB, ssB)
        s_wait(dB, rowsB, ssB)
        plsc.subcore_barrier()

    with jax.named_scope("agg_writeout"):
        pltpu.sync_copy(
            hagg.at[pl.ds(sid * ROWS_PER_TILE, ROWS_PER_TILE)],
            out_hbm.at[cid, pl.ds(sid * ROWS_PER_TILE, ROWS_PER_TILE)])


def _linear_relu_body(parts_ref, wt_ref, b_ref, o_ref):
    x = parts_ref[0] + parts_ref[1]
    y = jnp.dot(x, wt_ref[...], preferred_element_type=jnp.float32)
    o_ref[...] = jnp.maximum(y + b_ref[...], 0.0)


_BLK = 2000


def _apply_linear_relu(parts, wt, b2):
    return pl.pallas_call(
        _linear_relu_body,
        grid=(N_NODES // _BLK,),
        in_specs=[
            pl.BlockSpec((NC, _BLK, D), lambda i: (0, i, 0)),
            pl.BlockSpec((D, D), lambda i: (0, 0)),
            pl.BlockSpec((1, D), lambda i: (0, 0)),
        ],
        out_specs=pl.BlockSpec((_BLK, D), lambda i: (i, 0)),
        out_shape=jax.ShapeDtypeStruct((N_NODES, D), jnp.float32),
    )(parts, wt, b2)


@jax.jit
def kernel(feature, edge_index, W, b):
    src = edge_index[0].astype(jnp.int32)
    dst = edge_index[1].astype(jnp.int32)
    pad = E_PAD - N_EDGES
    idx = jnp.arange(pad, dtype=jnp.int32)
    src_p = jnp.concatenate([src, idx % 128])
    dst_p = jnp.concatenate([dst, N_NODES + idx % (N_PAD - N_NODES)])
    packed = jnp.bitwise_or(
        src_p, lax.shift_left(dst_p, 16)).reshape(NW, G, GROUP)
    zeros = jnp.zeros((N_PAD, D), jnp.float32)
    parts = _gcn_aggregate(feature, packed, zeros)
    return _apply_linear_relu(parts, W.T, b.reshape(1, D))
